# baseline (speedup 1.0000x reference)
import functools

import jax
import jax.numpy as jnp
from jax import lax
from jax.experimental import pallas as pl
from jax.experimental.pallas import tpu as pltpu
from jax.experimental.pallas import tpu_sc as plsc

H = 16
D = 64
K = 64
HALF_S = 16
S = 2 * HALF_S + 1
MAX_F = 16.0
MIN_F = 1.0
HALO = int(HALF_S * MAX_F + MAX_F)

NC = 2
NS = 16
NW = NC * NS

BL = 256
W = BL + 2 * HALO

_RNE_MAGIC = 12582912.0


def _silu(v):
    return v * jax.nn.sigmoid(v)


def _proj_body(x_ref, wkT_ref, bk_ref, wwT_ref, bw_ref,
               kern_ref, freq_ref, phase_ref):
    xb = x_ref[...]
    kern_ref[...] = _silu(
        jnp.dot(xb, wkT_ref[...], preferred_element_type=jnp.float32)
        + bk_ref[...])
    wave = _silu(
        jnp.dot(xb, wwT_ref[...], preferred_element_type=jnp.float32)
        + bw_ref[...])
    freq_ref[...] = jax.nn.sigmoid(wave[:, :H]) * (MAX_F - MIN_F) + MIN_F
    phase_ref[...] = jnp.tanh(wave[:, H:]) * MAX_F


def _projections(x2d, WkT, bk, WwT, bw, BM):
    M, C = x2d.shape
    grid = (M // BM,)
    return pl.pallas_call(
        _proj_body,
        grid=grid,
        in_specs=[
            pl.BlockSpec((BM, C), lambda i: (i, 0)),
            pl.BlockSpec((C, H * K), lambda i: (0, 0)),
            pl.BlockSpec((1, H * K), lambda i: (0, 0)),
            pl.BlockSpec((C, 2 * H), lambda i: (0, 0)),
            pl.BlockSpec((1, 2 * H), lambda i: (0, 0)),
        ],
        out_specs=[
            pl.BlockSpec((BM, H * K), lambda i: (i, 0)),
            pl.BlockSpec((BM, H), lambda i: (i, 0)),
            pl.BlockSpec((BM, H), lambda i: (i, 0)),
        ],
        out_shape=[
            jax.ShapeDtypeStruct((M, H * K), jnp.float32),
            jax.ShapeDtypeStruct((M, H), jnp.float32),
            jax.ShapeDtypeStruct((M, H), jnp.float32),
        ],
    )(x2d, WkT, bk, WwT, bw)


def _out_body(h_ref, woT_ref, o_ref):
    o_ref[...] = _silu(
        jnp.dot(h_ref[...], woT_ref[...], preferred_element_type=jnp.float32))


def _out_proj(h2d, WoT, BM):
    M, C = h2d.shape
    return pl.pallas_call(
        _out_body,
        grid=(M // BM,),
        in_specs=[
            pl.BlockSpec((BM, C), lambda i: (i, 0)),
            pl.BlockSpec((C, C), lambda i: (0, 0)),
        ],
        out_specs=pl.BlockSpec((BM, C), lambda i: (i, 0)),
        out_shape=jax.ShapeDtypeStruct((M, C), jnp.float32),
    )(h2d, WoT)


def _gconv_body(L, nb, xt_ref, fq_ref, ph_ref, kw_ref, hid_ref,
                win, fqv, phv, kwv, outv):
    wid = lax.axis_index("s") * NC + lax.axis_index("c")

    def block(blk, carry):
        l0 = blk * BL
        s0 = jnp.clip(l0 - HALO, 0, L - W)
        woff = pl.multiple_of(wid * (L * D) + s0 * D, 128)
        pltpu.sync_copy(xt_ref.at[pl.ds(woff, W * D)], win)
        boff = pl.multiple_of((wid * nb + blk) * BL, 256)
        pltpu.sync_copy(fq_ref.at[pl.ds(boff, BL)], fqv)
        pltpu.sync_copy(ph_ref.at[pl.ds(boff, BL)], phv)
        koff = pl.multiple_of((wid * nb + blk) * (S * BL), 128)
        pltpu.sync_copy(kw_ref.at[pl.ds(koff, S * BL)], kwv)

        def chunk(c, carry2):
            c16 = c * 16
            lvec = lax.iota(jnp.int32, 16).astype(jnp.float32) + (
                (l0 + c16).astype(jnp.float32))
            f = fqv[pl.ds(c16, 16)]
            p = phv[pl.ds(c16, 16)]
            base = lvec + p

            def sample(s):
                pos = base + jnp.float32(s - HALF_S) * f
                r = (pos + _RNE_MAGIC) - _RNE_MAGIC
                r = jnp.clip(r, 0.0, jnp.float32(L - 1))
                il = r.astype(jnp.int32) - s0
                return il * D, kwv[pl.ds(s * BL + c16, 16)]

            idxA = [sample(s) for s in range(17)]

            def dlA(d, cA):
                acc = jnp.zeros((16,), jnp.float32)
                for fl, w in idxA:
                    acc = acc + w * plsc.load_gather(win, [fl + d])
                outv[pl.ds(d * BL + c16, 16)] = acc
                return cA

            lax.fori_loop(0, D, dlA, 0, unroll=2)

            idxB = [sample(s) for s in range(17, S)]

            def dlB(d, cB):
                acc = jnp.zeros((16,), jnp.float32)
                for fl, w in idxB:
                    acc = acc + w * plsc.load_gather(win, [fl + d])
                plsc.addupdate(outv.at[pl.ds(d * BL + c16, 16)], acc)
                return cB

            lax.fori_loop(0, D, dlB, 0, unroll=2)
            return carry2

        lax.fori_loop(0, BL // 16, chunk, 0)
        ooff = pl.multiple_of((wid * nb + blk) * (D * BL), 256)
        pltpu.sync_copy(outv, hid_ref.at[pl.ds(ooff, D * BL)])
        return carry

    lax.fori_loop(0, nb, block, 0)


def _gather_conv(xt_flat, fqt, pht, kwt, L, nb):
    BH = xt_flat.shape[0] // (L * D)
    mesh = plsc.VectorSubcoreMesh(core_axis_name="c", subcore_axis_name="s",
                                  num_cores=NC, num_subcores=NS)
    k = pl.kernel(
        functools.partial(_gconv_body, L, nb),
        out_type=jax.ShapeDtypeStruct((BH * nb * D * BL,), jnp.float32),
        mesh=mesh,
        scratch_types=[
            pltpu.VMEM((W * D,), jnp.float32),
            pltpu.VMEM((BL,), jnp.float32),
            pltpu.VMEM((BL,), jnp.float32),
            pltpu.VMEM((S * BL,), jnp.float32),
            pltpu.VMEM((D * BL,), jnp.float32),
        ],
        compiler_params=pltpu.CompilerParams(needs_layout_passes=False),
    )
    return k(xt_flat, fqt, pht, kwt)


def kernel(x, Ww, bw, Wk, bk, Wo):
    B, L, C = x.shape
    M = B * L
    BH = B * H
    nb = L // BL

    x2d = x.reshape(M, C)
    kern2d, freq2d, phase2d = _projections(
        x2d, Wk.T, bk[None, :], Ww.T, bw[None, :], BM=512)

    xt_flat = (x.reshape(B, L, H, D).transpose(0, 2, 1, 3)
               .reshape(BH * L * D))
    fqt = (freq2d.reshape(B, L, H).transpose(0, 2, 1)
           .reshape(BH * nb * BL))
    pht = (phase2d.reshape(B, L, H).transpose(0, 2, 1)
           .reshape(BH * nb * BL))
    kwt = (kern2d.reshape(B, L, H, K)[:, :, :, :S]
           .reshape(B, nb, BL, H, S).transpose(0, 3, 1, 4, 2)
           .reshape(BH * nb * S * BL))

    hid = _gather_conv(xt_flat, fqt, pht, kwt, L, nb)

    h2d = (hid.reshape(B, H, nb, D, BL).transpose(0, 2, 4, 1, 3)
           .reshape(M, C))
    out2d = _out_proj(h2d, Wo.T, BM=512)
    return out2d.reshape(B, L, C)


# speedup vs baseline: 1.1063x; 1.1063x over previous
import functools

import jax
import jax.numpy as jnp
from jax import lax
from jax.experimental import pallas as pl
from jax.experimental.pallas import tpu as pltpu
from jax.experimental.pallas import tpu_sc as plsc

H = 16
D = 64
K = 64
HALF_S = 16
S = 2 * HALF_S + 1
MAX_F = 16.0
MIN_F = 1.0
HALO = int(HALF_S * MAX_F + MAX_F)

NC = 2
NS = 16
NW = NC * NS

BL = 256
W = BL + 2 * HALO

_RNE_MAGIC = 12582912.0


def _silu(v):
    return v * jax.nn.sigmoid(v)


def _proj_body(x_ref, wkT_ref, bk_ref, wwT_ref, bw_ref,
               kern_ref, freq_ref, phase_ref):
    xb = x_ref[...]
    kern_ref[...] = _silu(
        jnp.dot(xb, wkT_ref[...], preferred_element_type=jnp.float32)
        + bk_ref[...])
    wave = _silu(
        jnp.dot(xb, wwT_ref[...], preferred_element_type=jnp.float32)
        + bw_ref[...])
    freq_ref[...] = jax.nn.sigmoid(wave[:, :H]) * (MAX_F - MIN_F) + MIN_F
    phase_ref[...] = jnp.tanh(wave[:, H:]) * MAX_F


def _projections(x2d, WkT, bk, WwT, bw, BM):
    M, C = x2d.shape
    grid = (M // BM,)
    return pl.pallas_call(
        _proj_body,
        grid=grid,
        in_specs=[
            pl.BlockSpec((BM, C), lambda i: (i, 0)),
            pl.BlockSpec((C, H * K), lambda i: (0, 0)),
            pl.BlockSpec((1, H * K), lambda i: (0, 0)),
            pl.BlockSpec((C, 2 * H), lambda i: (0, 0)),
            pl.BlockSpec((1, 2 * H), lambda i: (0, 0)),
        ],
        out_specs=[
            pl.BlockSpec((BM, H * K), lambda i: (i, 0)),
            pl.BlockSpec((BM, H), lambda i: (i, 0)),
            pl.BlockSpec((BM, H), lambda i: (i, 0)),
        ],
        out_shape=[
            jax.ShapeDtypeStruct((M, H * K), jnp.float32),
            jax.ShapeDtypeStruct((M, H), jnp.float32),
            jax.ShapeDtypeStruct((M, H), jnp.float32),
        ],
    )(x2d, WkT, bk, WwT, bw)


def _out_body(h_ref, woT_ref, o_ref):
    o_ref[...] = _silu(
        jnp.dot(h_ref[...], woT_ref[...], preferred_element_type=jnp.float32))


def _out_proj(h2d, WoT, BM):
    M, C = h2d.shape
    return pl.pallas_call(
        _out_body,
        grid=(M // BM,),
        in_specs=[
            pl.BlockSpec((BM, C), lambda i: (i, 0)),
            pl.BlockSpec((C, C), lambda i: (0, 0)),
        ],
        out_specs=pl.BlockSpec((BM, C), lambda i: (i, 0)),
        out_shape=jax.ShapeDtypeStruct((M, C), jnp.float32),
    )(h2d, WoT)


def _gconv_body(L, nb, xt_ref, fq_ref, ph_ref, kw_ref, hid_ref,
                win, fqv, phv, kwv, outv):
    wid = lax.axis_index("s") * NC + lax.axis_index("c")

    def block(blk, carry):
        l0 = blk * BL
        s0 = jnp.clip(l0 - HALO, 0, L - W)
        woff = pl.multiple_of(wid * (L * D) + s0 * D, 128)
        pltpu.sync_copy(xt_ref.at[pl.ds(woff, W * D)], win)
        boff = pl.multiple_of((wid * nb + blk) * BL, 256)
        pltpu.sync_copy(fq_ref.at[pl.ds(boff, BL)], fqv)
        pltpu.sync_copy(ph_ref.at[pl.ds(boff, BL)], phv)
        koff = pl.multiple_of((wid * nb + blk) * (S * BL), 128)
        pltpu.sync_copy(kw_ref.at[pl.ds(koff, S * BL)], kwv)

        def chunk(c, carry2):
            c16 = c * 16
            lvec = lax.iota(jnp.int32, 16).astype(jnp.float32) + (
                (l0 + c16).astype(jnp.float32))
            f = fqv[pl.ds(c16, 16)]
            p = phv[pl.ds(c16, 16)]
            base = lvec + p

            def sample(s):
                pos = base + jnp.float32(s - HALF_S) * f
                r = (pos + _RNE_MAGIC) - _RNE_MAGIC
                r = jnp.clip(r, 0.0, jnp.float32(L - 1))
                il = r.astype(jnp.int32) - s0
                return il * D, kwv[pl.ds(s * BL + c16, 16)]

            def tree_sum(vs):
                while len(vs) > 1:
                    nxt = [vs[i] + vs[i + 1] for i in range(0, len(vs) - 1, 2)]
                    if len(vs) % 2:
                        nxt.append(vs[-1])
                    vs = nxt
                return vs[0]

            idxA = [sample(s) for s in range(17)]

            @plsc.parallel_loop(0, D, unroll=4)
            def dlA(d):
                acc = tree_sum(
                    [w * plsc.load_gather(win, [fl + d]) for fl, w in idxA])
                outv[pl.ds(d * BL + c16, 16)] = acc

            idxB = [sample(s) for s in range(17, S)]

            @plsc.parallel_loop(0, D, unroll=4)
            def dlB(d):
                acc = tree_sum(
                    [w * plsc.load_gather(win, [fl + d]) for fl, w in idxB])
                plsc.addupdate(outv.at[pl.ds(d * BL + c16, 16)], acc)

            return carry2

        lax.fori_loop(0, BL // 16, chunk, 0)
        ooff = pl.multiple_of((wid * nb + blk) * (D * BL), 256)
        pltpu.sync_copy(outv, hid_ref.at[pl.ds(ooff, D * BL)])
        return carry

    lax.fori_loop(0, nb, block, 0)


def _gather_conv(xt_flat, fqt, pht, kwt, L, nb):
    BH = xt_flat.shape[0] // (L * D)
    mesh = plsc.VectorSubcoreMesh(core_axis_name="c", subcore_axis_name="s",
                                  num_cores=NC, num_subcores=NS)
    k = pl.kernel(
        functools.partial(_gconv_body, L, nb),
        out_type=jax.ShapeDtypeStruct((BH * nb * D * BL,), jnp.float32),
        mesh=mesh,
        scratch_types=[
            pltpu.VMEM((W * D,), jnp.float32),
            pltpu.VMEM((BL,), jnp.float32),
            pltpu.VMEM((BL,), jnp.float32),
            pltpu.VMEM((S * BL,), jnp.float32),
            pltpu.VMEM((D * BL,), jnp.float32),
        ],
        compiler_params=pltpu.CompilerParams(needs_layout_passes=False),
    )
    return k(xt_flat, fqt, pht, kwt)


def kernel(x, Ww, bw, Wk, bk, Wo):
    B, L, C = x.shape
    M = B * L
    BH = B * H
    nb = L // BL

    x2d = x.reshape(M, C)
    kern2d, freq2d, phase2d = _projections(
        x2d, Wk.T, bk[None, :], Ww.T, bw[None, :], BM=512)

    xt_flat = (x.reshape(B, L, H, D).transpose(0, 2, 1, 3)
               .reshape(BH * L * D))
    fqt = (freq2d.reshape(B, L, H).transpose(0, 2, 1)
           .reshape(BH * nb * BL))
    pht = (phase2d.reshape(B, L, H).transpose(0, 2, 1)
           .reshape(BH * nb * BL))
    kwt = (kern2d.reshape(B, L, H, K)[:, :, :, :S]
           .reshape(B, nb, BL, H, S).transpose(0, 3, 1, 4, 2)
           .reshape(BH * nb * S * BL))

    hid = _gather_conv(xt_flat, fqt, pht, kwt, L, nb)

    h2d = (hid.reshape(B, H, nb, D, BL).transpose(0, 2, 4, 1, 3)
           .reshape(M, C))
    out2d = _out_proj(h2d, Wo.T, BM=512)
    return out2d.reshape(B, L, C)


# speedup vs baseline: 3.8768x; 3.5044x over previous
import functools

import jax
import jax.numpy as jnp
from jax import lax
from jax.experimental import pallas as pl
from jax.experimental.pallas import tpu as pltpu
from jax.experimental.pallas import tpu_sc as plsc

H = 16
D = 64
K = 64
HALF_S = 16
S = 2 * HALF_S + 1
MAX_F = 16.0
MIN_F = 1.0
HALO = int(HALF_S * MAX_F + MAX_F)

NC = 2
NS = 16
NW = NC * NS

BL = 256
W = BL + 2 * HALO + 8
DP = D + 1

_RNE_MAGIC = 12582912.0


def _silu(v):
    return v * jax.nn.sigmoid(v)


def _proj_body(x_ref, wkT_ref, bk_ref, wwT_ref, bw_ref,
               kern_ref, freq_ref, phase_ref):
    xb = x_ref[...]
    kern_ref[...] = _silu(
        jnp.dot(xb, wkT_ref[...], preferred_element_type=jnp.float32)
        + bk_ref[...])
    wave = _silu(
        jnp.dot(xb, wwT_ref[...], preferred_element_type=jnp.float32)
        + bw_ref[...])
    freq_ref[...] = jax.nn.sigmoid(wave[:, :H]) * (MAX_F - MIN_F) + MIN_F
    phase_ref[...] = jnp.tanh(wave[:, H:]) * MAX_F


def _projections(x2d, WkT, bk, WwT, bw, BM):
    M, C = x2d.shape
    grid = (M // BM,)
    return pl.pallas_call(
        _proj_body,
        grid=grid,
        in_specs=[
            pl.BlockSpec((BM, C), lambda i: (i, 0)),
            pl.BlockSpec((C, H * K), lambda i: (0, 0)),
            pl.BlockSpec((1, H * K), lambda i: (0, 0)),
            pl.BlockSpec((C, 2 * H), lambda i: (0, 0)),
            pl.BlockSpec((1, 2 * H), lambda i: (0, 0)),
        ],
        out_specs=[
            pl.BlockSpec((BM, H * K), lambda i: (i, 0)),
            pl.BlockSpec((BM, H), lambda i: (i, 0)),
            pl.BlockSpec((BM, H), lambda i: (i, 0)),
        ],
        out_shape=[
            jax.ShapeDtypeStruct((M, H * K), jnp.float32),
            jax.ShapeDtypeStruct((M, H), jnp.float32),
            jax.ShapeDtypeStruct((M, H), jnp.float32),
        ],
    )(x2d, WkT, bk, WwT, bw)


def _out_body(h_ref, woT_ref, o_ref):
    o_ref[...] = _silu(
        jnp.dot(h_ref[...], woT_ref[...], preferred_element_type=jnp.float32))


def _out_proj(h2d, WoT, BM):
    M, C = h2d.shape
    return pl.pallas_call(
        _out_body,
        grid=(M // BM,),
        in_specs=[
            pl.BlockSpec((BM, C), lambda i: (i, 0)),
            pl.BlockSpec((C, C), lambda i: (0, 0)),
        ],
        out_specs=pl.BlockSpec((BM, C), lambda i: (i, 0)),
        out_shape=jax.ShapeDtypeStruct((M, C), jnp.float32),
    )(h2d, WoT)


def _gconv_body(L, nb, xt_ref, fq_ref, ph_ref, kw_ref, hid_ref,
                win, fqv, phv, kwv, outv):
    wid = lax.axis_index("s") * NC + lax.axis_index("c")

    def block(blk, carry):
        l0 = blk * BL
        s0 = jnp.clip(l0 - HALO, 0, L - W) & ~7
        woff = pl.multiple_of(wid * (L * DP) + s0 * DP, 8)
        pltpu.sync_copy(xt_ref.at[pl.ds(woff, W * DP)], win)
        boff = pl.multiple_of((wid * nb + blk) * BL, 256)
        pltpu.sync_copy(fq_ref.at[pl.ds(boff, BL)], fqv)
        pltpu.sync_copy(ph_ref.at[pl.ds(boff, BL)], phv)
        koff = pl.multiple_of((wid * nb + blk) * (S * BL), 128)
        pltpu.sync_copy(kw_ref.at[pl.ds(koff, S * BL)], kwv)

        def chunk(c, carry2):
            c16 = c * 16
            lvec = lax.iota(jnp.int32, 16).astype(jnp.float32) + (
                (l0 + c16).astype(jnp.float32))
            f = fqv[pl.ds(c16, 16)]
            p = phv[pl.ds(c16, 16)]
            base = lvec + p

            def sample(s):
                pos = base + jnp.float32(s - HALF_S) * f
                r = (pos + _RNE_MAGIC) - _RNE_MAGIC
                r = jnp.clip(r, 0.0, jnp.float32(L - 1))
                il = r.astype(jnp.int32) - s0
                return il * DP, kwv[pl.ds(s * BL + c16, 16)]

            def tree_sum(vs):
                while len(vs) > 1:
                    nxt = [vs[i] + vs[i + 1] for i in range(0, len(vs) - 1, 2)]
                    if len(vs) % 2:
                        nxt.append(vs[-1])
                    vs = nxt
                return vs[0]

            idxA = [sample(s) for s in range(17)]

            @plsc.parallel_loop(0, D, unroll=4)
            def dlA(d):
                acc = tree_sum(
                    [w * plsc.load_gather(win, [fl + d]) for fl, w in idxA])
                outv[pl.ds(d * BL + c16, 16)] = acc

            idxB = [sample(s) for s in range(17, S)]

            @plsc.parallel_loop(0, D, unroll=4)
            def dlB(d):
                acc = tree_sum(
                    [w * plsc.load_gather(win, [fl + d]) for fl, w in idxB])
                plsc.addupdate(outv.at[pl.ds(d * BL + c16, 16)], acc)

            return carry2

        lax.fori_loop(0, BL // 16, chunk, 0)
        ooff = pl.multiple_of((wid * nb + blk) * (D * BL), 256)
        pltpu.sync_copy(outv, hid_ref.at[pl.ds(ooff, D * BL)])
        return carry

    lax.fori_loop(0, nb, block, 0)


def _gather_conv(xt_flat, fqt, pht, kwt, L, nb):
    BH = xt_flat.shape[0] // (L * DP)
    mesh = plsc.VectorSubcoreMesh(core_axis_name="c", subcore_axis_name="s",
                                  num_cores=NC, num_subcores=NS)
    k = pl.kernel(
        functools.partial(_gconv_body, L, nb),
        out_type=jax.ShapeDtypeStruct((BH * nb * D * BL,), jnp.float32),
        mesh=mesh,
        scratch_types=[
            pltpu.VMEM((W * DP,), jnp.float32),
            pltpu.VMEM((BL,), jnp.float32),
            pltpu.VMEM((BL,), jnp.float32),
            pltpu.VMEM((S * BL,), jnp.float32),
            pltpu.VMEM((D * BL,), jnp.float32),
        ],
        compiler_params=pltpu.CompilerParams(needs_layout_passes=False),
    )
    return k(xt_flat, fqt, pht, kwt)


def kernel(x, Ww, bw, Wk, bk, Wo):
    B, L, C = x.shape
    M = B * L
    BH = B * H
    nb = L // BL

    x2d = x.reshape(M, C)
    kern2d, freq2d, phase2d = _projections(
        x2d, Wk.T, bk[None, :], Ww.T, bw[None, :], BM=512)

    xt = x.reshape(B, L, H, D).transpose(0, 2, 1, 3)
    xt_flat = jnp.pad(xt, ((0, 0), (0, 0), (0, 0), (0, 1))).reshape(BH * L * DP)
    fqt = (freq2d.reshape(B, L, H).transpose(0, 2, 1)
           .reshape(BH * nb * BL))
    pht = (phase2d.reshape(B, L, H).transpose(0, 2, 1)
           .reshape(BH * nb * BL))
    kwt = (kern2d.reshape(B, L, H, K)[:, :, :, :S]
           .reshape(B, nb, BL, H, S).transpose(0, 3, 1, 4, 2)
           .reshape(BH * nb * S * BL))

    hid = _gather_conv(xt_flat, fqt, pht, kwt, L, nb)

    h2d = (hid.reshape(B, H, nb, D, BL).transpose(0, 2, 4, 1, 3)
           .reshape(M, C))
    out2d = _out_proj(h2d, Wo.T, BM=512)
    return out2d.reshape(B, L, C)
